# baseline (speedup 1.0000x reference)
import functools

import jax
import jax.numpy as jnp
from jax import lax
from jax.experimental import pallas as pl
from jax.experimental.pallas import tpu as pltpu
from jax.experimental.pallas import tpu_sc as plsc

B, L, H = 2, 2048, 768
F = 3072
E = 8
TOP_K = 2

T = B * L
A = TOP_K * T
BTF = 512
NBLK = A // BTF + E
S = NBLK * BTF

NC, NS = 2, 16
NW = NC * NS
APW = A // NW
SUB = 64
CH = 512


def _gate_body(x_ref, wg_ref, bg_ref, slot_ref, wsc_ref, eob_ref,
               m_ref, rank_ref):
    xb = x_ref[...]
    logits = jnp.dot(xb, wg_ref[...],
                     preferred_element_type=jnp.float32) + bg_ref[...][None, :]
    cols = lax.broadcasted_iota(jnp.int32, (T, E), 1)
    m0 = jnp.max(logits, axis=-1, keepdims=True)
    i0 = jnp.min(jnp.where(logits == m0, cols, E), axis=-1, keepdims=True)
    masked = jnp.where(cols == i0, -jnp.inf, logits)
    m1 = jnp.max(masked, axis=-1, keepdims=True)
    i1 = jnp.min(jnp.where(masked == m1, cols, E), axis=-1, keepdims=True)
    t = jnp.exp(m1 - m0)
    s0 = 1.0 / (1.0 + t)
    s1 = t / (1.0 + t)
    wsc_ref[pl.ds(0, T)] = jnp.reshape(s0, (T,))
    wsc_ref[pl.ds(T, T)] = jnp.reshape(s1, (T,))

    m_ref[pl.ds(0, T), :] = (cols == i0).astype(jnp.float32)
    m_ref[pl.ds(T, T), :] = (cols == i1).astype(jnp.float32)

    r = lax.broadcasted_iota(jnp.int32, (CH, CH), 0)
    c = lax.broadcasted_iota(jnp.int32, (CH, CH), 1)
    lstrict = (r > c).astype(jnp.float32)

    def body(ci, carry):
        blk = m_ref[pl.ds(ci * CH, CH), :]
        rank_ref[pl.ds(ci * CH, CH), :] = (
            jnp.dot(lstrict, blk, preferred_element_type=jnp.float32) + carry)
        return carry + jnp.sum(blk, axis=0, keepdims=True)

    counts = lax.fori_loop(0, A // CH, body, jnp.zeros((1, E), jnp.float32))

    nblk = jnp.ceil(counts / BTF)
    ri = lax.broadcasted_iota(jnp.int32, (E, E), 0)
    cj = lax.broadcasted_iota(jnp.int32, (E, E), 1)
    uincl = (ri <= cj).astype(jnp.float32)
    cb_incl = jnp.dot(nblk, uincl, preferred_element_type=jnp.float32)
    base_rows = (cb_incl - nblk) * BTF

    m = m_ref[...]
    rank_at = jnp.sum(rank_ref[...] * m, axis=1)
    base_at = jnp.sum(m * base_rows, axis=1)
    slot_ref[...] = (rank_at + base_at).astype(jnp.int32)

    bi = lax.broadcasted_iota(jnp.int32, (NBLK, E), 0)
    filled = (bi >= cb_incl.astype(jnp.int32)).astype(jnp.int32)
    eob_ref[pl.ds(0, NBLK)] = jnp.minimum(jnp.sum(filled, axis=1), E - 1)
    eob_ref[pl.ds(NBLK, 1)] = jnp.sum(nblk, axis=1).astype(jnp.int32)


def _gate(x2, Wg, bg, interpret=False):
    return pl.pallas_call(
        _gate_body,
        grid=(1,),
        in_specs=[
            pl.BlockSpec((T, H), lambda i: (0, 0)),
            pl.BlockSpec((H, E), lambda i: (0, 0)),
            pl.BlockSpec((E,), lambda i: (0,)),
        ],
        out_specs=[
            pl.BlockSpec((A,), lambda i: (0,)),
            pl.BlockSpec((A,), lambda i: (0,)),
            pl.BlockSpec((NBLK + 1,), lambda i: (0,)),
        ],
        out_shape=[
            jax.ShapeDtypeStruct((A,), jnp.int32),
            jax.ShapeDtypeStruct((A,), jnp.float32),
            jax.ShapeDtypeStruct((NBLK + 1,), jnp.int32),
        ],
        scratch_shapes=[
            pltpu.VMEM((A, E), jnp.float32),
            pltpu.VMEM((A, E), jnp.float32),
        ],
        interpret=interpret,
    )(x2, Wg, bg)


def _make_scatter():
    mesh = plsc.VectorSubcoreMesh(core_axis_name="c", subcore_axis_name="s",
                                  num_cores=NC, num_subcores=NS)

    nch = APW // SUB

    @functools.partial(
        pl.kernel, mesh=mesh,
        out_type=[
            jax.ShapeDtypeStruct((S, H), jnp.float32),
            jax.ShapeDtypeStruct((S,), jnp.float32),
        ],
        scratch_types=[
            pltpu.VMEM((nch, SUB), jnp.int32),
            pltpu.VMEM((nch, SUB), jnp.float32),
            pltpu.VMEM((2, SUB, H), jnp.float32),
            pltpu.SemaphoreType.DMA,
            pltpu.SemaphoreType.DMA,
            pltpu.SemaphoreType.DMA,
            pltpu.SemaphoreType.DMA,
            pltpu.SemaphoreType.DMA,
        ],
    )
    def scatter_k(x_hbm, slot2_hbm, wsc2_hbm, xg_hbm, ws_hbm,
                  idx_v, wv_v, rows_v, ls0, ls1, ss0, ss1, wsem):
        wid = lax.axis_index("s") * NC + lax.axis_index("c")
        base = wid * APW
        pltpu.sync_copy(slot2_hbm.at[pl.ds(wid * nch, nch), :], idx_v)
        pltpu.sync_copy(wsc2_hbm.at[pl.ds(wid * nch, nch), :], wv_v)
        wh = [pltpu.async_copy(wv_v.at[j], ws_hbm.at[idx_v.at[j]], wsem)
              for j in range(nch)]
        lsem = [ls0, ls1]
        ssem = [ss0, ss1]

        def load(j):
            t0 = lax.rem(base + j * SUB, T)
            return pltpu.async_copy(x_hbm.at[pl.ds(t0, SUB), :],
                                    rows_v.at[j % 2], lsem[j % 2])

        def scat(j):
            return pltpu.async_copy(rows_v.at[j % 2], xg_hbm.at[idx_v.at[j]],
                                    ssem[j % 2])

        l0 = load(0)
        l1 = load(1)
        l0.wait()
        s0 = scat(0)
        l1.wait()
        s1 = scat(1)
        s0.wait()
        l2 = load(2)
        s1.wait()
        l3 = load(3)
        l2.wait()
        s2 = scat(2)
        l3.wait()
        s3 = scat(3)
        s2.wait()
        s3.wait()
        for h in wh:
            h.wait()

    return scatter_k


def _ffn_body(eob_ref, xg_ref, ws_ref, w1_ref, b1_ref, w2_ref, b2_ref,
              yg_ref):
    g = pl.program_id(0)

    @pl.when(g < eob_ref[NBLK])
    def _active():
        h = jnp.dot(xg_ref[...], w1_ref[0],
                    preferred_element_type=jnp.float32)
        h = h + b1_ref[0]
        h = 0.5 * h * (1.0 + lax.erf(h * 0.7071067811865476))
        yg_ref[...] = (jnp.dot(h, w2_ref[0],
                               preferred_element_type=jnp.float32)
                       + b2_ref[0]) * ws_ref[...]


def _ffn(eob, xg, ws, W1, b1r, W2, b2r, interpret=False):
    return pl.pallas_call(
        _ffn_body,
        grid_spec=pltpu.PrefetchScalarGridSpec(
            num_scalar_prefetch=1,
            grid=(NBLK,),
            in_specs=[
                pl.BlockSpec((BTF, H), lambda g, eob: (g, 0)),
                pl.BlockSpec((BTF, 1), lambda g, eob: (g, 0)),
                pl.BlockSpec((1, H, F), lambda g, eob: (eob[g], 0, 0)),
                pl.BlockSpec((1, 1, F), lambda g, eob: (eob[g], 0, 0)),
                pl.BlockSpec((1, F, H), lambda g, eob: (eob[g], 0, 0)),
                pl.BlockSpec((1, 1, H), lambda g, eob: (eob[g], 0, 0)),
            ],
            out_specs=pl.BlockSpec((BTF, H), lambda g, eob: (g, 0)),
        ),
        out_shape=jax.ShapeDtypeStruct((S, H), jnp.float32),
        compiler_params=pltpu.CompilerParams(
            dimension_semantics=("arbitrary",),
        ),
        interpret=interpret,
    )(eob, xg, ws, W1, b1r, W2, b2r)


def _make_gather():
    mesh = plsc.VectorSubcoreMesh(core_axis_name="c", subcore_axis_name="s",
                                  num_cores=NC, num_subcores=NS)

    TPW = T // NW
    CHT = 32
    nchk = TPW // CHT

    @functools.partial(
        pl.kernel, mesh=mesh,
        out_type=jax.ShapeDtypeStruct((T, H), jnp.float32),
        scratch_types=[
            pltpu.VMEM((4, SUB), jnp.int32),
            pltpu.VMEM((2, 2, CHT, H), jnp.float32),
            pltpu.SemaphoreType.DMA,
            pltpu.SemaphoreType.DMA,
            pltpu.SemaphoreType.DMA,
            pltpu.SemaphoreType.DMA,
        ],
    )
    def gather_k(yg_hbm, slot2_hbm, out_hbm, idx_v, rows_v, g0s, g1s,
                 s0s, s1s):
        wid = lax.axis_index("s") * NC + lax.axis_index("c")
        rb0 = wid * (TPW // SUB)
        pltpu.sync_copy(slot2_hbm.at[pl.ds(rb0, 2), :],
                        idx_v.at[pl.ds(0, 2), :])
        pltpu.sync_copy(slot2_hbm.at[pl.ds(T // SUB + rb0, 2), :],
                        idx_v.at[pl.ds(2, 2), :])
        gsem = [g0s, g1s]
        ssem = [s0s, s1s]

        def gath(c):
            b = c % 2
            row = c // 2
            col = (c % 2) * CHT
            h0 = pltpu.async_copy(yg_hbm.at[idx_v.at[row, pl.ds(col, CHT)]],
                                  rows_v.at[b, 0], gsem[b])
            h1 = pltpu.async_copy(yg_hbm.at[idx_v.at[2 + row,
                                                     pl.ds(col, CHT)]],
                                  rows_v.at[b, 1], gsem[b])
            return h0, h1

        def adds(c):
            b = c % 2

            def rbody(r, carry):
                for v in range(H // 16):
                    sl = pl.ds(v * 16, 16)
                    rows_v[b, 0, r, sl] = rows_v[b, 0, r, sl] + \
                        rows_v[b, 1, r, sl]
                return carry

            lax.fori_loop(0, CHT, rbody, 0)

        def store(c):
            b = c % 2
            t0 = wid * TPW + c * CHT
            return pltpu.async_copy(rows_v.at[b, 0],
                                    out_hbm.at[pl.ds(t0, CHT), :], ssem[b])

        gs = {0: gath(0), 1: gath(1)}
        sts = {}
        for c in range(nchk):
            gs[c][0].wait()
            gs[c][1].wait()
            adds(c)
            sts[c] = store(c)
            if c + 2 < nchk:
                sts[c].wait()
                gs[c + 2] = gath(c + 2)
        sts[nchk - 2].wait()
        sts[nchk - 1].wait()

    return gather_k


_BC = 512


def _combine_body(z0_ref, z1_ref, sc_ref, out_ref):
    s = sc_ref[...]
    out_ref[...] = s[:, 0:1] * z0_ref[...] + s[:, 1:2] * z1_ref[...]


def _combine(z, scores, interpret=False):
    return pl.pallas_call(
        _combine_body,
        grid=(T // _BC,),
        in_specs=[
            pl.BlockSpec((_BC, H), lambda i: (i, 0)),
            pl.BlockSpec((_BC, H), lambda i: (i + T // _BC, 0)),
            pl.BlockSpec((_BC, TOP_K), lambda i: (i, 0)),
        ],
        out_specs=pl.BlockSpec((_BC, H), lambda i: (i, 0)),
        out_shape=jax.ShapeDtypeStruct((T, H), jnp.float32),
        interpret=interpret,
    )(z, z, scores)


_make_scatter = functools.cache(_make_scatter)
_make_gather = functools.cache(_make_gather)


@jax.jit
def kernel(x, Wg, bg, W1, b1, W2, b2):
    x2 = x.reshape(T, H)
    slot, wsc, eob = _gate(x2, Wg, bg)
    slot2 = slot.reshape(A // SUB, SUB)
    wsc2 = wsc.reshape(A // SUB, SUB)
    xg, ws = _make_scatter()(x2, slot2, wsc2)
    yg = _ffn(eob, xg, ws.reshape(S, 1), W1, b1.reshape(E, 1, F), W2,
              b2.reshape(E, 1, H))
    out = _make_gather()(yg, slot2)
    return out.reshape(B, L, H)


# speedup vs baseline: 1.1203x; 1.1203x over previous
import functools

import jax
import jax.numpy as jnp
from jax import lax
from jax.experimental import pallas as pl
from jax.experimental.pallas import tpu as pltpu
from jax.experimental.pallas import tpu_sc as plsc

B, L, H = 2, 2048, 768
F = 3072
E = 8
TOP_K = 2

T = B * L
A = TOP_K * T
BTF = 512
NBLK = A // BTF + E
S = NBLK * BTF

NC, NS = 2, 16
NW = NC * NS
APW = A // NW
SUB = 64
CH = 512


def _gate_body(x_ref, wg_ref, bg_ref, slot_ref, scores_ref, eob_ref,
               m_ref, rank_ref):
    xb = x_ref[...]
    logits = jnp.dot(xb, wg_ref[...],
                     preferred_element_type=jnp.float32) + bg_ref[...][None, :]
    cols = lax.broadcasted_iota(jnp.int32, (T, E), 1)
    m0 = jnp.max(logits, axis=-1, keepdims=True)
    i0 = jnp.min(jnp.where(logits == m0, cols, E), axis=-1, keepdims=True)
    masked = jnp.where(cols == i0, -jnp.inf, logits)
    m1 = jnp.max(masked, axis=-1, keepdims=True)
    i1 = jnp.min(jnp.where(masked == m1, cols, E), axis=-1, keepdims=True)
    t = jnp.exp(m1 - m0)
    s0 = 1.0 / (1.0 + t)
    s1 = t / (1.0 + t)
    scores_ref[...] = jnp.concatenate([s0, s1], axis=1)

    m_ref[pl.ds(0, T), :] = (cols == i0).astype(jnp.float32)
    m_ref[pl.ds(T, T), :] = (cols == i1).astype(jnp.float32)

    r = lax.broadcasted_iota(jnp.int32, (CH, CH), 0)
    c = lax.broadcasted_iota(jnp.int32, (CH, CH), 1)
    lstrict = (r > c).astype(jnp.float32)

    def body(ci, carry):
        blk = m_ref[pl.ds(ci * CH, CH), :]
        rank_ref[pl.ds(ci * CH, CH), :] = (
            jnp.dot(lstrict, blk, preferred_element_type=jnp.float32) + carry)
        return carry + jnp.sum(blk, axis=0, keepdims=True)

    counts = lax.fori_loop(0, A // CH, body, jnp.zeros((1, E), jnp.float32))

    nblk = jnp.ceil(counts / BTF)
    ri = lax.broadcasted_iota(jnp.int32, (E, E), 0)
    cj = lax.broadcasted_iota(jnp.int32, (E, E), 1)
    uincl = (ri <= cj).astype(jnp.float32)
    cb_incl = jnp.dot(nblk, uincl, preferred_element_type=jnp.float32)
    base_rows = (cb_incl - nblk) * BTF

    m = m_ref[...]
    rank_at = jnp.sum(rank_ref[...] * m, axis=1)
    base_at = jnp.sum(m * base_rows, axis=1)
    slot_ref[...] = (rank_at + base_at).astype(jnp.int32)

    bi = lax.broadcasted_iota(jnp.int32, (NBLK, E), 0)
    filled = (bi >= cb_incl.astype(jnp.int32)).astype(jnp.int32)
    eob_ref[pl.ds(0, NBLK)] = jnp.minimum(jnp.sum(filled, axis=1), E - 1)
    eob_ref[pl.ds(NBLK, 1)] = jnp.sum(nblk, axis=1).astype(jnp.int32)


def _gate(x2, Wg, bg, interpret=False):
    return pl.pallas_call(
        _gate_body,
        grid=(1,),
        in_specs=[
            pl.BlockSpec((T, H), lambda i: (0, 0)),
            pl.BlockSpec((H, E), lambda i: (0, 0)),
            pl.BlockSpec((E,), lambda i: (0,)),
        ],
        out_specs=[
            pl.BlockSpec((A,), lambda i: (0,)),
            pl.BlockSpec((T, TOP_K), lambda i: (0, 0)),
            pl.BlockSpec((NBLK + 1,), lambda i: (0,)),
        ],
        out_shape=[
            jax.ShapeDtypeStruct((A,), jnp.int32),
            jax.ShapeDtypeStruct((T, TOP_K), jnp.float32),
            jax.ShapeDtypeStruct((NBLK + 1,), jnp.int32),
        ],
        scratch_shapes=[
            pltpu.VMEM((A, E), jnp.float32),
            pltpu.VMEM((A, E), jnp.float32),
        ],
        interpret=interpret,
    )(x2, Wg, bg)


def _make_scatter():
    mesh = plsc.VectorSubcoreMesh(core_axis_name="c", subcore_axis_name="s",
                                  num_cores=NC, num_subcores=NS)

    nch = APW // SUB

    @functools.partial(
        pl.kernel, mesh=mesh,
        out_type=jax.ShapeDtypeStruct((S, H), jnp.float32),
        scratch_types=[
            pltpu.VMEM((nch, SUB), jnp.int32),
            pltpu.VMEM((2, SUB, H), jnp.float32),
            pltpu.SemaphoreType.DMA,
            pltpu.SemaphoreType.DMA,
            pltpu.SemaphoreType.DMA,
            pltpu.SemaphoreType.DMA,
        ],
    )
    def scatter_k(x_hbm, slot2_hbm, xg_hbm, idx_v, rows_v, ls0, ls1, ss0, ss1):
        wid = lax.axis_index("s") * NC + lax.axis_index("c")
        base = wid * APW
        pltpu.sync_copy(slot2_hbm.at[pl.ds(wid * nch, nch), :], idx_v)
        lsem = [ls0, ls1]
        ssem = [ss0, ss1]

        def load(j):
            t0 = lax.rem(base + j * SUB, T)
            return pltpu.async_copy(x_hbm.at[pl.ds(t0, SUB), :],
                                    rows_v.at[j % 2], lsem[j % 2])

        def scat(j):
            return pltpu.async_copy(rows_v.at[j % 2], xg_hbm.at[idx_v.at[j]],
                                    ssem[j % 2])

        l0 = load(0)
        l1 = load(1)
        l0.wait()
        s0 = scat(0)
        l1.wait()
        s1 = scat(1)
        s0.wait()
        l2 = load(2)
        s1.wait()
        l3 = load(3)
        l2.wait()
        s2 = scat(2)
        l3.wait()
        s3 = scat(3)
        s2.wait()
        s3.wait()

    return scatter_k


def _ffn_body(eob_ref, xg_ref, w1_ref, b1_ref, w2_ref, b2_ref, yg_ref):
    g = pl.program_id(0)

    @pl.when(g < eob_ref[NBLK])
    def _active():
        h = jnp.dot(xg_ref[...], w1_ref[0],
                    preferred_element_type=jnp.float32)
        h = h + b1_ref[0]
        h = 0.5 * h * (1.0 + lax.erf(h * 0.7071067811865476))
        yg_ref[...] = (jnp.dot(h, w2_ref[0],
                               preferred_element_type=jnp.float32)
                       + b2_ref[0])


def _ffn(eob, xg, W1, b1r, W2, b2r, interpret=False):
    return pl.pallas_call(
        _ffn_body,
        grid_spec=pltpu.PrefetchScalarGridSpec(
            num_scalar_prefetch=1,
            grid=(NBLK,),
            in_specs=[
                pl.BlockSpec((BTF, H), lambda g, eob: (g, 0)),
                pl.BlockSpec((1, H, F), lambda g, eob: (eob[g], 0, 0)),
                pl.BlockSpec((1, 1, F), lambda g, eob: (eob[g], 0, 0)),
                pl.BlockSpec((1, F, H), lambda g, eob: (eob[g], 0, 0)),
                pl.BlockSpec((1, 1, H), lambda g, eob: (eob[g], 0, 0)),
            ],
            out_specs=pl.BlockSpec((BTF, H), lambda g, eob: (g, 0)),
        ),
        out_shape=jax.ShapeDtypeStruct((S, H), jnp.float32),
        compiler_params=pltpu.CompilerParams(
            dimension_semantics=("arbitrary",),
        ),
        interpret=interpret,
    )(eob, xg, W1, b1r, W2, b2r)


def _make_gather():
    mesh = plsc.VectorSubcoreMesh(core_axis_name="c", subcore_axis_name="s",
                                  num_cores=NC, num_subcores=NS)

    nch = APW // SUB

    @functools.partial(
        pl.kernel, mesh=mesh,
        out_type=jax.ShapeDtypeStruct((A, H), jnp.float32),
        scratch_types=[
            pltpu.VMEM((nch, SUB), jnp.int32),
            pltpu.VMEM((2, SUB, H), jnp.float32),
            pltpu.SemaphoreType.DMA,
            pltpu.SemaphoreType.DMA,
            pltpu.SemaphoreType.DMA,
            pltpu.SemaphoreType.DMA,
        ],
    )
    def gather_k(yg_hbm, slot2_hbm, z_hbm, idx_v, rows_v, gs0, gs1, ws0, ws1):
        wid = lax.axis_index("s") * NC + lax.axis_index("c")
        base = wid * APW
        pltpu.sync_copy(slot2_hbm.at[pl.ds(wid * nch, nch), :], idx_v)
        gsem = [gs0, gs1]
        wsem = [ws0, ws1]

        def gath(j):
            return pltpu.async_copy(yg_hbm.at[idx_v.at[j]], rows_v.at[j % 2],
                                    gsem[j % 2])

        def store(j):
            a0 = base + j * SUB
            return pltpu.async_copy(rows_v.at[j % 2],
                                    z_hbm.at[pl.ds(a0, SUB), :], wsem[j % 2])

        g0 = gath(0)
        g1 = gath(1)
        g0.wait()
        w0 = store(0)
        g1.wait()
        w1 = store(1)
        w0.wait()
        g2 = gath(2)
        w1.wait()
        g3 = gath(3)
        g2.wait()
        w2 = store(2)
        g3.wait()
        w3 = store(3)
        w2.wait()
        w3.wait()

    return gather_k


_BC = 512


def _combine_body(z0_ref, z1_ref, sc_ref, out_ref):
    s = sc_ref[...]
    out_ref[...] = s[:, 0:1] * z0_ref[...] + s[:, 1:2] * z1_ref[...]


def _combine(z, scores, interpret=False):
    return pl.pallas_call(
        _combine_body,
        grid=(T // _BC,),
        in_specs=[
            pl.BlockSpec((_BC, H), lambda i: (i, 0)),
            pl.BlockSpec((_BC, H), lambda i: (i + T // _BC, 0)),
            pl.BlockSpec((_BC, TOP_K), lambda i: (i, 0)),
        ],
        out_specs=pl.BlockSpec((_BC, H), lambda i: (i, 0)),
        out_shape=jax.ShapeDtypeStruct((T, H), jnp.float32),
        interpret=interpret,
    )(z, z, scores)


_make_scatter = functools.cache(_make_scatter)
_make_gather = functools.cache(_make_gather)


@jax.jit
def kernel(x, Wg, bg, W1, b1, W2, b2):
    x2 = x.reshape(T, H)
    slot, scores, eob = _gate(x2, Wg, bg)
    slot2 = slot.reshape(A // SUB, SUB)
    xg = _make_scatter()(x2, slot2)
    yg = _ffn(eob, xg, W1, b1.reshape(E, 1, F), W2, b2.reshape(E, 1, H))
    z = _make_gather()(yg, slot2)
    out = _combine(z, scores)
    return out.reshape(B, L, H)
